# baseline (speedup 1.0000x reference)
import functools

import jax
import jax.numpy as jnp
from jax import lax
from jax.experimental import pallas as pl
from jax.experimental.pallas import tpu as pltpu
from jax.experimental.pallas import tpu_sc as plsc

VOCAB = 1000000
EMBED = 64

NC, NS = 2, 16
NW = NC * NS
LANES = 16
SLAB = 128

_FULL_SLABS = VOCAB // SLAB
_TAIL = VOCAB - _FULL_SLABS * SLAB
_BASE_SLABS = _FULL_SLABS // NW
_EXTRA = _FULL_SLABS - _BASE_SLABS * NW


def _worker_id():
    return lax.axis_index("s") * NC + lax.axis_index("c")


def _transpose_slab(src, dst, n_rows):
    rows_j = [j * LANES + lax.iota(jnp.int32, LANES) for j in range(EMBED // LANES)]
    for q in range(n_rows // 2):
        for h in range(2):
            col = jnp.full((LANES,), 2 * q + h, dtype=jnp.int32)
            for j in range(EMBED // LANES):
                val = plsc.load_gather(src, [rows_j[j], col])
                dst[q, pl.ds(64 * h + j * LANES, LANES)] = val


def _compact_kernel():
    mesh = plsc.VectorSubcoreMesh(core_axis_name="c", subcore_axis_name="s")

    @functools.partial(
        pl.kernel,
        out_type=jax.ShapeDtypeStruct((VOCAB // 2, 128), jnp.float32),
        mesh=mesh,
        scratch_types=[
            pltpu.VMEM((EMBED, SLAB), jnp.float32),
            pltpu.VMEM((EMBED, SLAB), jnp.float32),
            pltpu.VMEM((SLAB // 2, 128), jnp.float32),
            pltpu.VMEM((SLAB // 2, 128), jnp.float32),
            pltpu.SemaphoreType.DMA,
            pltpu.SemaphoreType.DMA,
            pltpu.SemaphoreType.DMA,
            pltpu.SemaphoreType.DMA,
        ],
        compiler_params=pltpu.CompilerParams(use_tc_tiling_on_sc=True,
                                            needs_layout_passes=False),
    )
    def body(tbl_t, tail_pairs, pairs, vin0, vin1, vout0, vout1,
             gi0, gi1, go0, go1):
        w = _worker_id()
        vins = (vin0, vin1)
        vouts = (vout0, vout1)
        gis = (gi0, gi1)
        gos = (go0, go1)
        start = w * _BASE_SLABS + jnp.minimum(w, _EXTRA)
        n_slabs = _BASE_SLABS + jnp.where(w < _EXTRA, 1, 0)

        def slab_in(buf, s):
            i0 = pl.multiple_of((start + s) * SLAB, SLAB)
            return pltpu.make_async_copy(
                tbl_t.at[:, pl.ds(i0, SLAB)], vins[buf], gis[buf])

        def slab_out(buf, s):
            r0 = pl.multiple_of((start + s) * (SLAB // 2), SLAB // 2)
            return pltpu.make_async_copy(
                vouts[buf], pairs.at[pl.ds(r0, SLAB // 2)], gos[buf])

        slab_in(0, 0).start()

        def step(s, _):
            cur = lax.rem(s, 2)
            for buf in range(2):
                @pl.when(cur == buf)
                def _():
                    slab_in(buf, s).wait()
                    @pl.when(s + 1 < n_slabs)
                    def _():
                        slab_in(1 - buf, s + 1).start()
                    @pl.when(s >= 2)
                    def _():
                        slab_out(buf, s - 2).wait()
                    _transpose_slab(vins[buf], vouts[buf], SLAB)
                    slab_out(buf, s).start()
            return 0

        lax.fori_loop(0, n_slabs, step, 0)
        slab_out(0, 0).wait()
        slab_out(1, 0).wait()

        @pl.when(w == NW - 1)
        def _():
            pltpu.sync_copy(tail_pairs, vout0.at[pl.ds(0, _TAIL // 2)])
            tail_out = pltpu.make_async_copy(
                vout0.at[pl.ds(0, _TAIL // 2)],
                pairs.at[pl.ds(_FULL_SLABS * SLAB // 2, _TAIL // 2)], go0)
            tail_out.start()
            tail_out.wait()

    return body


def _lookup_kernel(B: int, T: int):
    chunk = 128
    bblocks = B // chunk
    n_chunks_total = T * bblocks
    per_w = n_chunks_total // NW

    mesh = plsc.VectorSubcoreMesh(core_axis_name="c", subcore_axis_name="s")

    @functools.partial(
        pl.kernel,
        out_type=jax.ShapeDtypeStruct((T, EMBED, B), jnp.float32),
        mesh=mesh,
        scratch_types=[
            pltpu.VMEM((chunk,), jnp.int32),
            pltpu.VMEM((chunk,), jnp.int32),
            pltpu.VMEM((chunk,), jnp.int32),
            pltpu.VMEM((chunk,), jnp.int32),
            pltpu.VMEM((chunk, 128), jnp.float32),
            pltpu.VMEM((chunk, 128), jnp.float32),
            pltpu.VMEM((EMBED, chunk), jnp.float32),
            pltpu.VMEM((EMBED, chunk), jnp.float32),
            pltpu.VMEM((EMBED, T), jnp.float32),
            pltpu.SemaphoreType.DMA,
            pltpu.SemaphoreType.DMA,
            pltpu.SemaphoreType.DMA,
            pltpu.SemaphoreType.DMA,
            pltpu.SemaphoreType.DMA,
            pltpu.SemaphoreType.DMA,
        ],
        compiler_params=pltpu.CompilerParams(use_tc_tiling_on_sc=True,
                                            needs_layout_passes=False),
    )
    def body(idx_f, pairs, pos_t, out, i0v, i1v, c0v, c1v, g0v, g1v,
             t0v, t1v, pos_v, si0, si1, gg0, gg1, oo0, oo1):
        w = _worker_id()
        idxv = (i0v, i1v)
        colv = (c0v, c1v)
        gath = (g0v, g1v)
        trv = (t0v, t1v)
        isems = (si0, si1)
        gsems = (gg0, gg1)
        osems = (oo0, oo1)
        pltpu.sync_copy(pos_t, pos_v)
        g_base = w * per_w

        def coords(s):
            g = g_base + s
            return lax.div(g, bblocks), lax.rem(g, bblocks) * chunk

        def idx_in(buf, s):
            f0 = pl.multiple_of((g_base + s) * chunk, chunk)
            return pltpu.make_async_copy(
                idx_f.at[pl.ds(f0, chunk)], idxv[buf], isems[buf])

        def gather_desc(buf):
            return pltpu.make_async_copy(
                pairs.at[idxv[buf]], gath[buf], gsems[buf])

        def prep(buf):
            for m in range(chunk // LANES):
                sl = pl.ds(m * LANES, LANES)
                v = idxv[buf][sl]
                colv[buf][sl] = jnp.left_shift(jnp.bitwise_and(v, 1), 6)
                idxv[buf][sl] = jnp.right_shift(v, 1)

        def out_copy(buf, s):
            t, b0 = coords(s)
            b0 = pl.multiple_of(b0, chunk)
            return pltpu.make_async_copy(
                trv[buf], out.at[t, :, pl.ds(b0, chunk)], osems[buf])

        def transpose_add(buf, s):
            t, _ = coords(s)
            t_vec = jnp.full((LANES,), t, dtype=jnp.int32)
            rows_m = [m * LANES + lax.iota(jnp.int32, LANES)
                      for m in range(chunk // LANES)]
            colb_m = [colv[buf][pl.ds(m * LANES, LANES)]
                      for m in range(chunk // LANES)]
            for e in range(EMBED):
                e_vec = jnp.full((LANES,), e, dtype=jnp.int32)
                pos_e = plsc.load_gather(pos_v, [e_vec, t_vec])
                for m in range(chunk // LANES):
                    val = plsc.load_gather(gath[buf], [rows_m[m], colb_m[m] + e])
                    trv[buf][e, pl.ds(m * LANES, LANES)] = val + pos_e

        idx_in(0, 0).start()
        idx_in(0, 0).wait()
        prep(0)
        gather_desc(0).start()
        idx_in(1, 1).start()

        def step(s, _):
            cur = lax.rem(s, 2)
            for buf in range(2):
                @pl.when(cur == buf)
                def _():
                    nxt = 1 - buf
                    gather_desc(buf).wait()
                    @pl.when(s + 2 < per_w)
                    def _():
                        idx_in(buf, s + 2).start()
                    @pl.when(s + 1 < per_w)
                    def _():
                        idx_in(nxt, s + 1).wait()
                        prep(nxt)
                        gather_desc(nxt).start()
                    @pl.when(s >= 2)
                    def _():
                        out_copy(buf, s - 2).wait()
                    transpose_add(buf, s)
                    out_copy(buf, s).start()
            return 0

        lax.fori_loop(0, per_w, step, 0)
        out_copy(0, 0).wait()
        out_copy(1, 0).wait()

    return body


def kernel(idx, token_table, pos_table):
    B, Tv = idx.shape
    tail_pairs = token_table[_FULL_SLABS * SLAB:].reshape(_TAIL // 2, 128)
    pairs = _compact_kernel()(token_table.T, tail_pairs)
    out_t = _lookup_kernel(B, Tv)(
        idx.T.reshape(B * Tv).astype(jnp.int32), pairs, pos_table.T)
    return out_t.transpose(2, 0, 1)


# speedup vs baseline: 2.9191x; 2.9191x over previous
import functools

import jax
import jax.numpy as jnp
from jax import lax
from jax.experimental import pallas as pl
from jax.experimental.pallas import tpu as pltpu
from jax.experimental.pallas import tpu_sc as plsc

VOCAB = 1000000
EMBED = 64
HALF = VOCAB // 2

NC, NS = 2, 16
NW = NC * NS
LANES = 16

_TC_R = 2048


def _wide_kernel():
    grid = (VOCAB + _TC_R - 1) // _TC_R

    def body(x_ref, out_ref):
        xt = x_ref[...].T
        out_ref[...] = jnp.concatenate([xt, xt], axis=1)

    return pl.pallas_call(
        body,
        grid=(grid,),
        in_specs=[pl.BlockSpec((EMBED, _TC_R), lambda i: (0, i))],
        out_specs=pl.BlockSpec((_TC_R, 128), lambda i: (i, 0)),
        out_shape=jax.ShapeDtypeStruct((VOCAB, 128), jnp.float32),
    )


def _lookup_kernel(B: int, T: int):
    chunk = 128
    bblocks = B // chunk
    per_w = T * bblocks // NW

    mesh = plsc.VectorSubcoreMesh(core_axis_name="c", subcore_axis_name="s")

    @functools.partial(
        pl.kernel,
        out_type=jax.ShapeDtypeStruct((T, B, EMBED), jnp.float32),
        mesh=mesh,
        scratch_types=[
            pltpu.VMEM((chunk,), jnp.int32),
            pltpu.VMEM((chunk,), jnp.int32),
            pltpu.VMEM((chunk, 128), jnp.float32),
            pltpu.VMEM((chunk, 128), jnp.float32),
            pltpu.VMEM((chunk, EMBED), jnp.float32),
            pltpu.VMEM((chunk, EMBED), jnp.float32),
            pltpu.VMEM((T, EMBED), jnp.float32),
            pltpu.SemaphoreType.DMA,
            pltpu.SemaphoreType.DMA,
            pltpu.SemaphoreType.DMA,
            pltpu.SemaphoreType.DMA,
            pltpu.SemaphoreType.DMA,
            pltpu.SemaphoreType.DMA,
        ],
        compiler_params=pltpu.CompilerParams(use_tc_tiling_on_sc=True,
                                            needs_layout_passes=False),
    )
    def body(idx_f, pairs, pos, out, i0v, i1v, g0v, g1v,
             o0v, o1v, pos_v, si0, si1, gg0, gg1, oo0, oo1):
        w = lax.axis_index("s") * NC + lax.axis_index("c")
        idxv = (i0v, i1v)
        gath = (g0v, g1v)
        outv = (o0v, o1v)
        isems = (si0, si1)
        gsems = (gg0, gg1)
        osems = (oo0, oo1)
        pltpu.sync_copy(pos, pos_v)
        g_base = w * per_w

        def coords(s):
            g = g_base + s
            return lax.div(g, bblocks), lax.rem(g, bblocks) * chunk

        def idx_in(buf, s):
            f0 = pl.multiple_of((g_base + s) * chunk, chunk)
            return (pltpu.make_async_copy(
                idx_f.at[pl.ds(f0, chunk)], idxv[buf], isems[buf]),)

        def gather_desc(buf):
            return pltpu.make_async_copy(
                pairs.at[idxv[buf]], gath[buf], gsems[buf])

        def out_copy(buf, s):
            t, b0 = coords(s)
            b0 = pl.multiple_of(b0, chunk)
            return pltpu.make_async_copy(
                outv[buf], out.at[t, pl.ds(b0, chunk)], osems[buf])

        def select_add(buf, s):
            t, _ = coords(s)
            pv = [pos_v[t, pl.ds(j * LANES, LANES)]
                  for j in range(EMBED // LANES)]
            for k in range(chunk):
                for j in range(EMBED // LANES):
                    sl = pl.ds(j * LANES, LANES)
                    outv[buf][k, sl] = gath[buf][k, sl] + pv[j]

        for d in idx_in(0, 0):
            d.start()
        for d in idx_in(0, 0):
            d.wait()
        gather_desc(0).start()
        for d in idx_in(1, 1):
            d.start()

        def step(s, _):
            cur = lax.rem(s, 2)
            for buf in range(2):
                @pl.when(cur == buf)
                def _():
                    nxt = 1 - buf
                    gather_desc(buf).wait()
                    @pl.when(s + 2 < per_w)
                    def _():
                        for d in idx_in(buf, s + 2):
                            d.start()
                    @pl.when(s + 1 < per_w)
                    def _():
                        for d in idx_in(nxt, s + 1):
                            d.wait()
                        gather_desc(nxt).start()
                    @pl.when(s >= 2)
                    def _():
                        out_copy(buf, s - 2).wait()
                    select_add(buf, s)
                    out_copy(buf, s).start()
            return 0

        lax.fori_loop(0, per_w, step, 0)
        out_copy(0, 0).wait()
        out_copy(1, 0).wait()

    return body


def kernel(idx, token_table, pos_table):
    B, Tv = idx.shape
    tbl_t = token_table.T
    pairs = _wide_kernel()(tbl_t)
    out_t = _lookup_kernel(B, Tv)(
        idx.T.reshape(B * Tv).astype(jnp.int32), pairs, pos_table)
    return out_t.transpose(1, 0, 2)
